# baseline (speedup 1.0000x reference)
import functools

import jax
import jax.numpy as jnp
from jax import lax
from jax.experimental import pallas as pl
from jax.experimental.pallas import tpu as pltpu
from jax.experimental.pallas import tpu_sc as plsc

SCALE_ = 8.0

NC_ = 2
NS_ = 16
NW_ = NC_ * NS_
LANES_ = 16

CHUNK_ = 512


def _make_kernel(total, d):
    b_per_w = total // NW_
    nchunks = b_per_w // CHUNK_
    assert nchunks % 2 == 0 and nchunks >= 4
    mesh = plsc.VectorSubcoreMesh(core_axis_name="c", subcore_axis_name="s")

    vecs_per_row = d // LANES_

    def scale_buf(buf):
        def body(r, carry):
            for j in range(vecs_per_row):
                sl = pl.ds(j * LANES_, LANES_)
                buf[r, sl] = buf[r, sl] * SCALE_
            return carry

        lax.fori_loop(0, CHUNK_, body, 0)

    @functools.partial(
        pl.kernel,
        out_type=jax.ShapeDtypeStruct((total, d), jnp.float32),
        mesh=mesh,
        scratch_types=[
            pltpu.VMEM((CHUNK_,), jnp.int32),
            pltpu.VMEM((CHUNK_,), jnp.int32),
            pltpu.VMEM((CHUNK_, d), jnp.float32),
            pltpu.VMEM((CHUNK_, d), jnp.float32),
            pltpu.SemaphoreType.DMA,
            pltpu.SemaphoreType.DMA,
        ],
        compiler_params=pltpu.CompilerParams(use_tc_tiling_on_sc=False),
    )
    def emb_kernel(idx_hbm, table_hbm, out_hbm, i0, i1, r0, r1, s0, s1):
        wid = lax.axis_index("s") * NC_ + lax.axis_index("c")
        base = wid * b_per_w
        idx_bufs = (i0, i1)
        row_bufs = (r0, r1)
        sems = (s0, s1)

        def start_gather(g, b):
            pltpu.sync_copy(idx_hbm.at[pl.ds(base + g * CHUNK_, CHUNK_)],
                            idx_bufs[b])
            pltpu.async_copy(table_hbm.at[idx_bufs[b]], row_bufs[b], sems[b])

        def finish(g, b):
            pltpu.make_async_copy(table_hbm.at[idx_bufs[b]], row_bufs[b],
                                  sems[b]).wait()
            scale_buf(row_bufs[b])
            pltpu.sync_copy(row_bufs[b],
                            out_hbm.at[pl.ds(base + g * CHUNK_, CHUNK_)])

        start_gather(0, 0)
        start_gather(1, 1)

        def outer(g2, carry):
            for b in range(2):
                g = g2 * 2 + b
                finish(g, b)
                start_gather(g + 2, b)
            return carry

        lax.fori_loop(0, nchunks // 2 - 1, outer, 0)

        for b in range(2):
            finish(nchunks - 2 + b, b)

    return emb_kernel


@jax.jit
def kernel(tokens, table):
    b, l = tokens.shape
    total = b * l
    d = table.shape[1]
    flat = tokens.reshape(total).astype(jnp.int32)
    out = _make_kernel(total, d)(flat, table)
    return out.reshape(b, l, d)
